# Initial kernel scaffold; baseline (speedup 1.0000x reference)
import jax
import jax.numpy as jnp
from jax.experimental import pallas as pl


def kernel(frames):
    raise NotImplementedError("write your pallas kernel here")


# speedup vs baseline: 2.7668x; 2.7668x over previous
import jax
import jax.numpy as jnp
import numpy as np
from jax.experimental import pallas as pl
from jax.experimental.pallas import tpu as pltpu

_ALPHA = 4


def _body(off_ref, in_ref, fast_ref, slow_ref):
    x = in_ref[...]
    fast_ref[...] = x
    off = off_ref[pl.program_id(0)]
    slow_ref[...] = in_ref[:, pl.ds(off, 1)]


def kernel(frames):
    C, T, H, W = frames.shape
    n = T // _ALPHA
    idx = np.linspace(0.0, T - 1, n).astype(np.int32)
    offs = idx - _ALPHA * np.arange(n, dtype=np.int32)
    assert (offs >= 0).all() and (offs < _ALPHA).all()

    grid_spec = pltpu.PrefetchScalarGridSpec(
        num_scalar_prefetch=1,
        grid=(n,),
        in_specs=[
            pl.BlockSpec((C, _ALPHA, H, W), lambda j, off: (0, j, 0, 0)),
        ],
        out_specs=[
            pl.BlockSpec((C, _ALPHA, H, W), lambda j, off: (0, j, 0, 0)),
            pl.BlockSpec((C, 1, H, W), lambda j, off: (0, j, 0, 0)),
        ],
    )
    fast, slow = pl.pallas_call(
        _body,
        grid_spec=grid_spec,
        out_shape=[
            jax.ShapeDtypeStruct((C, T, H, W), frames.dtype),
            jax.ShapeDtypeStruct((C, n, H, W), frames.dtype),
        ],
    )(jnp.asarray(offs), frames)
    return (slow, fast)
